# Initial kernel scaffold; baseline (speedup 1.0000x reference)
import jax
import jax.numpy as jnp
from jax.experimental import pallas as pl


def kernel(x, edge_index, pseudo, weight, root_weight, att_weight, bias):
    raise NotImplementedError("write your pallas kernel here")


# speedup vs baseline: 2.6614x; 2.6614x over previous
import functools

import jax
import jax.numpy as jnp
from jax import lax
from jax.experimental import pallas as pl
from jax.experimental.pallas import tpu as pltpu
from jax.experimental.pallas import tpu_sc as plsc

N = 10000
E = 160000
C = 128
K5 = 5
NEG = 0.2

NW = 32
CHUNK = 128
CPW = 40
EP = NW * CPW * CHUNK
NROWS = EP // CHUNK
NP = 10240
AC = 144
BE = 512
NB = EP // BE

@functools.cache
def _mesh():
    return plsc.VectorSubcoreMesh(core_axis_name="c", subcore_axis_name="s",
                                  num_cores=2, num_subcores=16)


def _root_body(x_ref, rw_ref, aw1_ref, root_ref, ratt_ref):
    root = jnp.dot(x_ref[...], rw_ref[...], preferred_element_type=jnp.float32)
    root_ref[...] = root
    ratt_ref[...] = jnp.sum(root * aw1_ref[...], axis=1, keepdims=True)


def _root_call(x, rw, aw1):
    return pl.pallas_call(
        _root_body,
        out_shape=(jax.ShapeDtypeStruct((N, C), jnp.float32),
                   jax.ShapeDtypeStruct((N, 1), jnp.float32)),
    )(x, rw, aw1)


def _gather_body(src2d_hbm, dst2d_hbm, x_hbm, ra_hbm, xg_hbm, rad_hbm,
                 srci, dsti, rows, rat, rao, sem):
    cid = lax.axis_index("c")
    sid = lax.axis_index("s")
    wid = sid * 2 + cid
    pltpu.sync_copy(ra_hbm, rat)
    lane = lax.iota(jnp.int32, 16)

    def chunk(ci, carry):
        row = wid * CPW + ci
        base = row * CHUNK
        pltpu.sync_copy(src2d_hbm.at[row], srci)
        pltpu.sync_copy(dst2d_hbm.at[row], dsti)
        cp1 = pltpu.async_copy(x_hbm.at[srci], rows, sem)
        for g in range(CHUNK // 16):
            iv = dsti[pl.ds(g * 16, 16)]
            acc = jnp.zeros((16,), jnp.float32)
            for j in range(16):
                v = rat[pl.ds(iv[j], 16)]
                acc = jnp.where(lane == j, v[0], acc)
            rao[pl.ds(g * 16, 16)] = acc
        cp1.wait()
        pltpu.sync_copy(rows, xg_hbm.at[pl.ds(base, CHUNK), :])
        pltpu.sync_copy(rao, rad_hbm.at[row])
        return carry

    lax.fori_loop(0, CPW, chunk, 0)


@functools.cache
def _gather_k():
    return pl.kernel(
        _gather_body,
        mesh=_mesh(),
        out_type=(jax.ShapeDtypeStruct((EP, C), jnp.float32),
                  jax.ShapeDtypeStruct((NROWS, CHUNK), jnp.float32)),
        compiler_params=pltpu.CompilerParams(use_tc_tiling_on_sc=False),
        scratch_types=[
            pltpu.VMEM((CHUNK,), jnp.int32),
            pltpu.VMEM((CHUNK,), jnp.int32),
            pltpu.VMEM((CHUNK, C), jnp.float32),
            pltpu.VMEM((NP,), jnp.float32),
            pltpu.VMEM((CHUNK,), jnp.float32),
            pltpu.SemaphoreType.DMA,
        ],
    )


def _msg_body(xg_ref, pp_ref, ra_ref, wr_ref, aw2_ref, msg_ref, lgt_ref,
              gmx_ref):
    i = pl.program_id(0)
    xg = xg_ref[...]
    pp = pp_ref[...]
    v = pp * jnp.float32(K5 - 1)
    bot = jnp.floor(v)
    frac = v - bot
    b0 = jnp.clip(bot.astype(jnp.int32), 0, K5 - 1)
    b1 = jnp.clip(b0 + 1, 0, K5 - 1)
    r0, c0 = b0[:, 0:1], b0[:, 1:2]
    r1, c1 = b1[:, 0:1], b1[:, 1:2]
    fr, fc = frac[:, 0:1], frac[:, 1:2]

    xgb = xg.astype(jnp.bfloat16)
    cols = []
    for r in range(K5):
        t_r = (jnp.where(r0 == r, 1.0 - fr, 0.0)
               + jnp.where(r1 == r, fr, 0.0))
        cols.append(xgb * t_r.astype(jnp.bfloat16))
    xr = jnp.concatenate(cols, axis=1)
    m2 = jnp.dot(xr, wr_ref[...], preferred_element_type=jnp.float32)

    msg = jnp.zeros((BE, C), jnp.float32)
    for c in range(K5):
        u_c = (jnp.where(c0 == c, 1.0 - fc, 0.0)
               + jnp.where(c1 == c, fc, 0.0))
        msg = msg + u_c * m2[:, c * C:(c + 1) * C]
    msg_ref[...] = msg

    lg = jnp.sum(msg * aw2_ref[...], axis=1) + ra_ref[0, 0, :]
    lg = jnp.where(lg >= 0, lg, NEG * lg)
    lgt_ref[0, 0, :] = lg

    @pl.when(i == 0)
    def _():
        gmx_ref[...] = jnp.full((8, 128), -1e30, jnp.float32)

    gmx_ref[...] = jnp.maximum(gmx_ref[...], jnp.max(lg))


def _msg_call(xg, pp, rad3, wr, aw2):
    return pl.pallas_call(
        _msg_body,
        grid=(NB,),
        in_specs=[
            pl.BlockSpec((BE, C), lambda i: (i, 0)),
            pl.BlockSpec((BE, 2), lambda i: (i, 0)),
            pl.BlockSpec((1, 1, BE), lambda i: (i, 0, 0)),
            pl.BlockSpec((K5 * C, K5 * C), lambda i: (0, 0)),
            pl.BlockSpec((1, C), lambda i: (0, 0)),
        ],
        out_specs=[
            pl.BlockSpec((BE, C), lambda i: (i, 0)),
            pl.BlockSpec((1, 1, BE), lambda i: (i, 0, 0)),
            pl.BlockSpec((8, 128), lambda i: (0, 0)),
        ],
        out_shape=(jax.ShapeDtypeStruct((EP, C), jnp.float32),
                   jax.ShapeDtypeStruct((NB, 1, BE), jnp.float32),
                   jax.ShapeDtypeStruct((8, 128), jnp.float32)),
    )(xg, pp, rad3, wr, aw2)


def _exmsg_body(msg_ref, lgt_ref, gmx_ref, out_ref):
    gm = gmx_ref[0, 0]
    ex = jnp.exp(lgt_ref[0, 0, :] - gm)
    exc = ex[:, None]
    out_ref[:, 0:C] = msg_ref[...] * exc
    lane = lax.broadcasted_iota(jnp.int32, (BE, AC - C), 1)
    out_ref[:, C:AC] = jnp.where(lane == 0, exc, 0.0)


def _exmsg_call(msg, lgt, gmx):
    return pl.pallas_call(
        _exmsg_body,
        grid=(NB,),
        in_specs=[
            pl.BlockSpec((BE, C), lambda i: (i, 0)),
            pl.BlockSpec((1, 1, BE), lambda i: (i, 0, 0)),
            pl.BlockSpec((8, 128), lambda i: (0, 0)),
        ],
        out_specs=pl.BlockSpec((BE, AC), lambda i: (i, 0)),
        out_shape=jax.ShapeDtypeStruct((EP, AC), jnp.float32),
    )(msg, lgt, gmx)


def _scatter_body(dst2d_hbm, exmsg_hbm, out_hbm, dsti, rowsv, zbuf, acc, sem):
    cid = lax.axis_index("c")
    sid = lax.axis_index("s")
    wid = sid * 2 + cid

    zero = jnp.zeros((16,), jnp.float32)

    def zrow(e, carry):
        for kk in range(AC // 16):
            zbuf[e, pl.ds(kk * 16, 16)] = zero
        return carry

    lax.fori_loop(0, 64, zrow, 0)

    def zcp(k, carry):
        pltpu.sync_copy(zbuf, acc.at[pl.ds(sid * 640 + k * 64, 64), :])
        return carry

    lax.fori_loop(0, NP // (16 * 64), zcp, 0)
    plsc.subcore_barrier()

    def chunk(ci, carry):
        row = wid * CPW + ci
        base = row * CHUNK
        pltpu.sync_copy(dst2d_hbm.at[row], dsti)
        pltpu.sync_copy(exmsg_hbm.at[pl.ds(base, CHUNK), :], rowsv)
        pltpu.sync_copy(rowsv, acc.at[dsti], add=True)
        return carry

    lax.fori_loop(0, CPW, chunk, 0)
    plsc.subcore_barrier()

    def cp(k, carry):
        r0 = sid * 640 + k * 64
        pltpu.sync_copy(acc.at[pl.ds(r0, 64), :],
                        out_hbm.at[cid, pl.ds(r0, 64), :])
        return carry

    lax.fori_loop(0, NP // (16 * 64), cp, 0)


@functools.cache
def _scatter_k():
    return pl.kernel(
        _scatter_body,
        mesh=_mesh(),
        out_type=jax.ShapeDtypeStruct((2, NP, AC), jnp.float32),
        compiler_params=pltpu.CompilerParams(use_tc_tiling_on_sc=False),
        scratch_types=[
            pltpu.VMEM((CHUNK,), jnp.int32),
            pltpu.VMEM((CHUNK, AC), jnp.float32),
            pltpu.VMEM((64, AC), jnp.float32),
            pltpu.VMEM_SHARED((NP, AC), jnp.float32),
            pltpu.SemaphoreType.DMA,
        ],
    )


def _final_body(pacc_ref, root_ref, bias_ref, out_ref):
    a = pacc_ref[0] + pacc_ref[1]
    agg = a[:, 0:C]
    den = a[:, C:C + 1]
    out_ref[...] = agg / (den + 1e-16) + root_ref[...] + bias_ref[...]


def _final_call(pacc, root, bias2):
    return pl.pallas_call(
        _final_body,
        grid=(10,),
        in_specs=[
            pl.BlockSpec((2, 1000, AC), lambda i: (0, i, 0)),
            pl.BlockSpec((1000, C), lambda i: (i, 0)),
            pl.BlockSpec((1, C), lambda i: (0, 0)),
        ],
        out_specs=pl.BlockSpec((1000, C), lambda i: (i, 0)),
        out_shape=jax.ShapeDtypeStruct((N, C), jnp.float32),
    )(pacc, root, bias2)


def kernel(x, edge_index, pseudo, weight, root_weight, att_weight, bias):
    src = edge_index[0].astype(jnp.int32)
    dst = edge_index[1].astype(jnp.int32)
    pad = EP - E
    src2d = jnp.concatenate(
        [src, jnp.zeros((pad,), jnp.int32)]).reshape(NROWS, CHUNK)
    dst2d = jnp.concatenate(
        [dst, jnp.full((pad,), N, jnp.int32)]).reshape(NROWS, CHUNK)
    pp = jnp.concatenate([pseudo, jnp.zeros((pad, 2), jnp.float32)])
    wr = (weight.reshape(K5, K5, C, C).transpose(0, 2, 1, 3)
          .reshape(K5 * C, K5 * C).astype(jnp.bfloat16))
    aw1 = att_weight[:C].reshape(1, C)
    aw2 = att_weight[C:].reshape(1, C)

    root, ratt = _root_call(x, root_weight, aw1)
    ra_tab = jnp.concatenate([ratt[:, 0], jnp.zeros((NP - N,), jnp.float32)])

    xg, rad = _gather_k()(src2d, dst2d, x, ra_tab)
    rad3 = rad.reshape(NB, 1, BE)

    msg, lgt, gmx = _msg_call(xg, pp, rad3, wr, aw2)
    exmsg = _exmsg_call(msg, lgt, gmx)
    pacc = _scatter_k()(dst2d, exmsg)
    return _final_call(pacc, root, bias.reshape(1, C))
